# Initial kernel scaffold; baseline (speedup 1.0000x reference)
import jax
import jax.numpy as jnp
from jax.experimental import pallas as pl


def kernel(node_attr, pos, edge_attr, params, edge_index, batch):
    raise NotImplementedError("write your pallas kernel here")


# speedup vs baseline: 1.9499x; 1.9499x over previous
import functools

import jax
import jax.numpy as jnp
from jax import lax
from jax.experimental import pallas as pl
from jax.experimental.pallas import tpu as pltpu
from jax.experimental.pallas import tpu_sc as plsc

N = 10000
E = 320000
HID = 128
EDIM = 16
NG = 64
XW = 16
C = HID + XW

NTILES = 32
EPT = E // NTILES
CHUNK = 80
NCH = EPT // CHUNK
NROW = N // 16

BE = 2000
BN = 2000


def _silu(x):
    return x * jax.nn.sigmoid(x)


def _sc_gather_body(t1_hbm, t2_hbm, dst3_hbm, src3_hbm, u1_hbm, u2_hbm,
                    idxd_v, idxs_v, u1_v, u2_v, sem1, sem2):
    wid = lax.axis_index("s") * 2 + lax.axis_index("c")
    pltpu.sync_copy(dst3_hbm.at[wid], idxd_v)
    pltpu.sync_copy(src3_hbm.at[wid], idxs_v)

    def body(k, carry):
        base = wid * EPT + k * CHUNK
        cp1 = pltpu.make_async_copy(t1_hbm.at[idxd_v.at[k]], u1_v, sem1)
        cp2 = pltpu.make_async_copy(t2_hbm.at[idxs_v.at[k]], u2_v, sem2)
        cp1.start()
        cp2.start()
        cp1.wait()
        cp2.wait()
        pltpu.sync_copy(u1_v, u1_hbm.at[pl.ds(base, CHUNK)])
        pltpu.sync_copy(u2_v, u2_hbm.at[pl.ds(base, CHUNK)])
        return carry

    lax.fori_loop(0, NCH, body, 0)


def _sc_scatter_body(v_hbm, dst3_hbm, zeros_hbm, p_hbm, acc_sh, idx_v, v_v):
    c = lax.axis_index("c")
    s = lax.axis_index("s")
    wid = s * 2 + c
    pltpu.sync_copy(zeros_hbm.at[pl.ds(s * NROW, NROW)],
                    acc_sh.at[pl.ds(s * NROW, NROW)])
    plsc.subcore_barrier()
    pltpu.sync_copy(dst3_hbm.at[wid], idx_v)

    def body(k, carry):
        base = wid * EPT + k * CHUNK
        pltpu.sync_copy(v_hbm.at[pl.ds(base, CHUNK)], v_v)
        pltpu.sync_copy(v_v, acc_sh.at[idx_v.at[k]], add=True)
        return carry

    lax.fori_loop(0, NCH, body, 0)
    plsc.subcore_barrier()
    pltpu.sync_copy(acc_sh.at[pl.ds(s * NROW, NROW)], p_hbm.at[c, s])


@functools.cache
def _sc_kernels():
    mesh = plsc.VectorSubcoreMesh(core_axis_name="c", subcore_axis_name="s")
    params = pltpu.CompilerParams(use_tc_tiling_on_sc=False)
    gather = pl.kernel(
        _sc_gather_body,
        compiler_params=params,
        out_type=[jax.ShapeDtypeStruct((E, C), jnp.float32),
                  jax.ShapeDtypeStruct((E, C), jnp.float32)],
        mesh=mesh,
        scratch_types=[pltpu.VMEM((NCH, CHUNK), jnp.int32),
                       pltpu.VMEM((NCH, CHUNK), jnp.int32),
                       pltpu.VMEM((CHUNK, C), jnp.float32),
                       pltpu.VMEM((CHUNK, C), jnp.float32),
                       pltpu.SemaphoreType.DMA,
                       pltpu.SemaphoreType.DMA],
    )
    scatter = pl.kernel(
        _sc_scatter_body,
        compiler_params=params,
        out_type=jax.ShapeDtypeStruct((2, 16, NROW, C), jnp.float32),
        mesh=mesh,
        scratch_types=[pltpu.VMEM_SHARED((N, C), jnp.float32),
                       pltpu.VMEM((NCH, CHUNK), jnp.int32),
                       pltpu.VMEM((CHUNK, C), jnp.float32)],
    )
    return gather, scatter


def _full(shape):
    return pl.BlockSpec(shape, lambda i: (0, 0))


def _rows(shape):
    return pl.BlockSpec(shape, lambda i: (i, 0))


def _dot(a, b):
    return jnp.dot(a, b, preferred_element_type=jnp.float32)


def _b(x):
    return x.astype(jnp.bfloat16).astype(jnp.float32)


def _node_init_body(na_ref, x_ref, wemb, bemb, wi, bi, wj,
                    h_ref, t1_ref, t2_ref):
    h = _dot(na_ref[...], wemb[...]) + bemb[...]
    h_ref[...] = h
    x = x_ref[...]
    t1_ref[:, :HID] = _dot(h, wi[...]) + bi[...]
    t1_ref[:, HID:] = x
    t2_ref[:, :HID] = _dot(h, wj[...])
    t2_ref[:, HID:] = x


def _edge_body(u1_ref, u2_ref, ea_ref, we, wr, we1, be1, winf, binf,
               wx0, bx0, wx1, bx1, v_ref):
    u1 = u1_ref[...]
    u2 = u2_ref[...]
    g = u1[:, :HID] + u2[:, :HID]
    d = u1[:, HID:] - u2[:, HID:]
    r2 = jnp.sum(d * d, axis=1, keepdims=True)
    pre = g + _b(r2) * _b(wr[...]) + _dot(ea_ref[...], we[...])
    u = _silu(pre)
    m1 = _silu(_dot(u, we1[...]) + be1[...])
    gate = jax.nn.sigmoid(
        jnp.sum(_b(m1) * _b(winf[...]), axis=1, keepdims=True) + binf[...])
    m = gate * m1
    t = _silu(_dot(m, wx0[...]) + bx0[...])
    coef = jnp.sum(_b(t) * _b(wx1[...]), axis=1, keepdims=True) + bx1[...]
    v_ref[:, :HID] = m
    v_ref[:, HID:] = d * coef


def _node_mid_body(h_ref, x_ref, p0_ref, p1_ref, wh0h, wh0m, bh0, wh1, bh1,
                   wi, bi, wj, hn_ref, xn_ref, t1_ref, t2_ref):
    h = h_ref[...]
    magg = p0_ref[:, :HID] + p1_ref[:, :HID]
    dx = p0_ref[:, HID:] + p1_ref[:, HID:]
    xn = x_ref[...] + dx
    u = _silu(_dot(h, wh0h[...]) + _dot(magg, wh0m[...]) + bh0[...])
    hn = _dot(u, wh1[...]) + bh1[...]
    hn_ref[...] = hn
    xn_ref[...] = xn
    t1_ref[:, :HID] = _dot(hn, wi[...]) + bi[...]
    t1_ref[:, HID:] = xn
    t2_ref[:, :HID] = _dot(hn, wj[...])
    t2_ref[:, HID:] = xn


def _node_last_body(h_ref, p0_ref, p1_ref, wh0h, wh0m, bh0, wh1, bh1,
                    hn_ref):
    h = h_ref[...]
    magg = p0_ref[:, :HID] + p1_ref[:, :HID]
    u = _silu(_dot(h, wh0h[...]) + _dot(magg, wh0m[...]) + bh0[...])
    hn_ref[...] = _dot(u, wh1[...]) + bh1[...]


def _readout_body(h_ref, b_ref, w0, b0, w1, b1, wp0, bp0, wp1, bp1,
                  sums_ref, cnts_ref, out_ref):
    i = pl.program_id(0)

    @pl.when(i == 0)
    def _():
        sums_ref[...] = jnp.zeros_like(sums_ref)
        cnts_ref[...] = jnp.zeros_like(cnts_ref)
        out_ref[...] = jnp.zeros_like(out_ref)

    t = _silu(_dot(h_ref[...], w0[...]) + b0[...])
    t = _dot(t, w1[...]) + b1[...]
    og = (b_ref[...] == lax.broadcasted_iota(jnp.int32, (BN, NG), 1)
          ).astype(jnp.float32)
    cdims = (((0,), (0,)), ((), ()))
    sums_ref[...] += lax.dot_general(og, t, cdims,
                                     preferred_element_type=jnp.float32,
                                     precision=lax.Precision.HIGHEST)
    cnts_ref[...] += lax.dot_general(og, jnp.ones((BN, HID), jnp.float32),
                                     cdims, preferred_element_type=jnp.float32,
                                     precision=lax.Precision.HIGHEST)

    @pl.when(i == pl.num_programs(0) - 1)
    def _():
        hg = sums_ref[...] / jnp.maximum(cnts_ref[...], 1.0)
        z = _silu(_dot(hg, wp0[...]) + bp0[...])
        out_ref[...] = (jnp.sum(z * wp1[...], axis=1, keepdims=True)
                        + bp1[...])


def _node_init_call(na, x16, wemb, bemb, wi, bi, wj):
    grid = (N // BN,)
    return pl.pallas_call(
        _node_init_body,
        grid=grid,
        in_specs=[_rows((BN, HID)), _rows((BN, XW)),
                  _full((HID, HID)), _full((1, HID)),
                  _full((HID, HID)), _full((1, HID)), _full((HID, HID))],
        out_specs=[_rows((BN, HID)), _rows((BN, C)), _rows((BN, C))],
        out_shape=[jax.ShapeDtypeStruct((N, HID), jnp.float32),
                   jax.ShapeDtypeStruct((N, C), jnp.float32),
                   jax.ShapeDtypeStruct((N, C), jnp.float32)],
    )(na, x16, wemb, bemb, wi, bi, wj)


def _edge_call(u1, u2, ea, w):
    grid = (E // BE,)
    return pl.pallas_call(
        _edge_body,
        grid=grid,
        in_specs=[_rows((BE, C)), _rows((BE, C)), _rows((BE, EDIM)),
                  _full((EDIM, HID)), _full((1, HID)),
                  _full((HID, HID)), _full((1, HID)),
                  _full((1, HID)), _full((1, 1)),
                  _full((HID, HID)), _full((1, HID)),
                  _full((1, HID)), _full((1, 1))],
        out_specs=[_rows((BE, C))],
        out_shape=[jax.ShapeDtypeStruct((E, C), jnp.float32)],
    )(u1, u2, ea, w["we"], w["wr"], w["we1"], w["be1"], w["winf"], w["binf"],
      w["wx0"], w["bx0"], w["wx1"], w["bx1"])[0]


def _node_mid_call(h, x16, p0, p1, w, wi, bi, wj):
    grid = (N // BN,)
    return pl.pallas_call(
        _node_mid_body,
        grid=grid,
        in_specs=[_rows((BN, HID)), _rows((BN, XW)),
                  _rows((BN, C)), _rows((BN, C)),
                  _full((HID, HID)), _full((HID, HID)), _full((1, HID)),
                  _full((HID, HID)), _full((1, HID)),
                  _full((HID, HID)), _full((1, HID)), _full((HID, HID))],
        out_specs=[_rows((BN, HID)), _rows((BN, XW)),
                   _rows((BN, C)), _rows((BN, C))],
        out_shape=[jax.ShapeDtypeStruct((N, HID), jnp.float32),
                   jax.ShapeDtypeStruct((N, XW), jnp.float32),
                   jax.ShapeDtypeStruct((N, C), jnp.float32),
                   jax.ShapeDtypeStruct((N, C), jnp.float32)],
    )(h, x16, p0, p1, w["wh0h"], w["wh0m"], w["bh0"], w["wh1"], w["bh1"],
      wi, bi, wj)


def _node_last_call(h, p0, p1, w):
    grid = (N // BN,)
    return pl.pallas_call(
        _node_last_body,
        grid=grid,
        in_specs=[_rows((BN, HID)), _rows((BN, C)), _rows((BN, C)),
                  _full((HID, HID)), _full((HID, HID)), _full((1, HID)),
                  _full((HID, HID)), _full((1, HID))],
        out_specs=[_rows((BN, HID))],
        out_shape=[jax.ShapeDtypeStruct((N, HID), jnp.float32)],
    )(h, p0, p1, w["wh0h"], w["wh0m"], w["bh0"], w["wh1"], w["bh1"])[0]


def _readout_call(h, bids, w):
    grid = (N // BN,)
    return pl.pallas_call(
        _readout_body,
        grid=grid,
        in_specs=[_rows((BN, HID)), _rows((BN, 1)),
                  _full((HID, HID)), _full((1, HID)),
                  _full((HID, HID)), _full((1, HID)),
                  _full((HID, HID)), _full((1, HID)),
                  _full((1, HID)), _full((1, 1))],
        out_specs=[_full((NG, HID)), _full((NG, HID)), _full((NG, 1))],
        out_shape=[jax.ShapeDtypeStruct((NG, HID), jnp.float32),
                   jax.ShapeDtypeStruct((NG, HID), jnp.float32),
                   jax.ShapeDtypeStruct((NG, 1), jnp.float32)],
    )(h, bids, w["w0"], w["b0"], w["w1"], w["b1"],
      w["wp0"], w["bp0"], w["wp1"], w["bp1"])[2]


def _row(v):
    return v.reshape(1, -1)


def kernel(node_attr, pos, edge_attr, params, edge_index, batch):
    src = edge_index[0]
    dst = edge_index[1]
    x16 = jnp.pad(pos, ((0, 0), (0, XW - 3)))
    dst3 = dst.reshape(NTILES, NCH, CHUNK)
    src3 = src.reshape(NTILES, NCH, CHUNK)
    zeros = jnp.zeros((N, C), jnp.float32)
    bids = batch.reshape(N, 1)

    layers = params["layers"]

    def e0_split(lp):
        w = lp["e0"]["w"]
        return (w[:HID], _row(lp["e0"]["b"]), w[HID:2 * HID],
                w[2 * HID:2 * HID + 1], w[2 * HID + 1:])

    def edge_w(lp):
        _, _, _, wr, we = e0_split(lp)
        return {
            "we": we, "wr": wr,
            "we1": lp["e1"]["w"], "be1": _row(lp["e1"]["b"]),
            "winf": _row(lp["inf"]["w"][:, 0]),
            "binf": lp["inf"]["b"].reshape(1, 1),
            "wx0": lp["x0"]["w"], "bx0": _row(lp["x0"]["b"]),
            "wx1": _row(lp["x1"]["w"][:, 0]),
            "bx1": lp["x1"]["b"].reshape(1, 1),
        }

    def node_w(lp):
        return {
            "wh0h": lp["h0"]["w"][:HID], "wh0m": lp["h0"]["w"][HID:],
            "bh0": _row(lp["h0"]["b"]),
            "wh1": lp["h1"]["w"], "bh1": _row(lp["h1"]["b"]),
        }

    wi0, bi0, wj0, _, _ = e0_split(layers[0])
    h, t1, t2 = _node_init_call(
        node_attr, x16, params["embedding"]["w"],
        _row(params["embedding"]["b"]), wi0, bi0, wj0)

    sc_gather, sc_scatter = _sc_kernels()
    for l in range(len(layers)):
        lp = layers[l]
        u1, u2 = sc_gather(t1, t2, dst3, src3)
        v = _edge_call(u1, u2, edge_attr, edge_w(lp))
        p = sc_scatter(v, dst3, zeros).reshape(2, N, C)
        if l + 1 < len(layers):
            win, bin_, wjn, _, _ = e0_split(layers[l + 1])
            h, x16, t1, t2 = _node_mid_call(h, x16, p[0], p[1],
                                            node_w(lp), win, bin_, wjn)
        else:
            h = _node_last_call(h, p[0], p[1], node_w(lp))

    rw = {
        "w0": params["lin0"]["w"], "b0": _row(params["lin0"]["b"]),
        "w1": params["lin1"]["w"], "b1": _row(params["lin1"]["b"]),
        "wp0": params["pred0"]["w"], "bp0": _row(params["pred0"]["b"]),
        "wp1": _row(params["pred1"]["w"][:, 0]),
        "bp1": params["pred1"]["b"].reshape(1, 1),
    }
    out = _readout_call(h, bids, rw)
    return out.reshape(-1)


# speedup vs baseline: 1.9534x; 1.0018x over previous
import functools

import jax
import jax.numpy as jnp
from jax import lax
from jax.experimental import pallas as pl
from jax.experimental.pallas import tpu as pltpu
from jax.experimental.pallas import tpu_sc as plsc

N = 10000
E = 320000
HID = 128
EDIM = 16
NG = 64
XW = 16
C = HID + XW

NTILES = 32
EPT = E // NTILES
CHUNK = 80
NCH = EPT // CHUNK
NROW = N // 16

BE = 2000
BN = 2000


def _sigmoid(x):
    return 0.5 * jnp.tanh(0.5 * x) + 0.5


def _silu(x):
    return x * _sigmoid(x)


def _sc_gather_body(t1_hbm, t2_hbm, dst3_hbm, src3_hbm, u1_hbm, u2_hbm,
                    idxd_v, idxs_v, u1_v, u2_v, sem1, sem2):
    wid = lax.axis_index("s") * 2 + lax.axis_index("c")
    pltpu.sync_copy(dst3_hbm.at[wid], idxd_v)
    pltpu.sync_copy(src3_hbm.at[wid], idxs_v)

    def body(k, carry):
        base = wid * EPT + k * CHUNK
        cp1 = pltpu.make_async_copy(t1_hbm.at[idxd_v.at[k]], u1_v, sem1)
        cp2 = pltpu.make_async_copy(t2_hbm.at[idxs_v.at[k]], u2_v, sem2)
        cp1.start()
        cp2.start()
        cp1.wait()
        cp2.wait()
        pltpu.sync_copy(u1_v, u1_hbm.at[pl.ds(base, CHUNK)])
        pltpu.sync_copy(u2_v, u2_hbm.at[pl.ds(base, CHUNK)])
        return carry

    lax.fori_loop(0, NCH, body, 0)


def _sc_scatter_body(v_hbm, dst3_hbm, zeros_hbm, p_hbm, acc_sh, idx_v, v_v):
    c = lax.axis_index("c")
    s = lax.axis_index("s")
    wid = s * 2 + c
    pltpu.sync_copy(zeros_hbm.at[pl.ds(s * NROW, NROW)],
                    acc_sh.at[pl.ds(s * NROW, NROW)])
    plsc.subcore_barrier()
    pltpu.sync_copy(dst3_hbm.at[wid], idx_v)

    def body(k, carry):
        base = wid * EPT + k * CHUNK
        pltpu.sync_copy(v_hbm.at[pl.ds(base, CHUNK)], v_v)
        pltpu.sync_copy(v_v, acc_sh.at[idx_v.at[k]], add=True)
        return carry

    lax.fori_loop(0, NCH, body, 0)
    plsc.subcore_barrier()
    pltpu.sync_copy(acc_sh.at[pl.ds(s * NROW, NROW)], p_hbm.at[c, s])


@functools.cache
def _sc_kernels():
    mesh = plsc.VectorSubcoreMesh(core_axis_name="c", subcore_axis_name="s")
    params = pltpu.CompilerParams(use_tc_tiling_on_sc=False)
    gather = pl.kernel(
        _sc_gather_body,
        compiler_params=params,
        out_type=[jax.ShapeDtypeStruct((E, C), jnp.float32),
                  jax.ShapeDtypeStruct((E, C), jnp.float32)],
        mesh=mesh,
        scratch_types=[pltpu.VMEM((NCH, CHUNK), jnp.int32),
                       pltpu.VMEM((NCH, CHUNK), jnp.int32),
                       pltpu.VMEM((CHUNK, C), jnp.float32),
                       pltpu.VMEM((CHUNK, C), jnp.float32),
                       pltpu.SemaphoreType.DMA,
                       pltpu.SemaphoreType.DMA],
    )
    scatter = pl.kernel(
        _sc_scatter_body,
        compiler_params=params,
        out_type=jax.ShapeDtypeStruct((2, 16, NROW, C), jnp.float32),
        mesh=mesh,
        scratch_types=[pltpu.VMEM_SHARED((N, C), jnp.float32),
                       pltpu.VMEM((NCH, CHUNK), jnp.int32),
                       pltpu.VMEM((CHUNK, C), jnp.float32)],
    )
    return gather, scatter


def _full(shape):
    return pl.BlockSpec(shape, lambda i: (0, 0))


def _rows(shape):
    return pl.BlockSpec(shape, lambda i: (i, 0))


def _dot(a, b):
    return jnp.dot(a, b, preferred_element_type=jnp.float32)


def _b(x):
    return x.astype(jnp.bfloat16).astype(jnp.float32)


def _node_init_body(na_ref, x_ref, wemb, bemb, wi, bi, wj,
                    h_ref, t1_ref, t2_ref):
    h = _dot(na_ref[...], wemb[...]) + bemb[...]
    h_ref[...] = h
    x = x_ref[...]
    t1_ref[:, :HID] = _dot(h, wi[...]) + bi[...]
    t1_ref[:, HID:] = x
    t2_ref[:, :HID] = _dot(h, wj[...])
    t2_ref[:, HID:] = x


def _edge_body(u1_ref, u2_ref, ea_ref, we, wr, we1, be1, winf, binf,
               wx0, bx0, wx1, bx1, v_ref):
    u1 = u1_ref[...]
    u2 = u2_ref[...]
    g = u1[:, :HID] + u2[:, :HID]
    d = u1[:, HID:] - u2[:, HID:]
    r2 = jnp.sum(d * d, axis=1, keepdims=True)
    pre = g + _b(r2) * _b(wr[...]) + _dot(ea_ref[...], we[...])
    u = _silu(pre)
    m1 = _silu(_dot(u, we1[...]) + be1[...])
    gate = _sigmoid(
        jnp.sum(_b(m1) * _b(winf[...]), axis=1, keepdims=True) + binf[...])
    m = gate * m1
    t = _silu(_dot(m, wx0[...]) + bx0[...])
    coef = jnp.sum(_b(t) * _b(wx1[...]), axis=1, keepdims=True) + bx1[...]
    v_ref[:, :HID] = m
    v_ref[:, HID:] = d * coef


def _node_mid_body(h_ref, x_ref, p0_ref, p1_ref, wh0h, wh0m, bh0, wh1, bh1,
                   wi, bi, wj, hn_ref, xn_ref, t1_ref, t2_ref):
    h = h_ref[...]
    magg = p0_ref[:, :HID] + p1_ref[:, :HID]
    dx = p0_ref[:, HID:] + p1_ref[:, HID:]
    xn = x_ref[...] + dx
    u = _silu(_dot(h, wh0h[...]) + _dot(magg, wh0m[...]) + bh0[...])
    hn = _dot(u, wh1[...]) + bh1[...]
    hn_ref[...] = hn
    xn_ref[...] = xn
    t1_ref[:, :HID] = _dot(hn, wi[...]) + bi[...]
    t1_ref[:, HID:] = xn
    t2_ref[:, :HID] = _dot(hn, wj[...])
    t2_ref[:, HID:] = xn


def _node_last_body(h_ref, p0_ref, p1_ref, wh0h, wh0m, bh0, wh1, bh1,
                    hn_ref):
    h = h_ref[...]
    magg = p0_ref[:, :HID] + p1_ref[:, :HID]
    u = _silu(_dot(h, wh0h[...]) + _dot(magg, wh0m[...]) + bh0[...])
    hn_ref[...] = _dot(u, wh1[...]) + bh1[...]


def _readout_body(h_ref, b_ref, w0, b0, w1, b1, wp0, bp0, wp1, bp1,
                  sums_ref, cnts_ref, out_ref):
    i = pl.program_id(0)

    @pl.when(i == 0)
    def _():
        sums_ref[...] = jnp.zeros_like(sums_ref)
        cnts_ref[...] = jnp.zeros_like(cnts_ref)
        out_ref[...] = jnp.zeros_like(out_ref)

    t = _silu(_dot(h_ref[...], w0[...]) + b0[...])
    t = _dot(t, w1[...]) + b1[...]
    og = (b_ref[...] == lax.broadcasted_iota(jnp.int32, (BN, NG), 1)
          ).astype(jnp.float32)
    cdims = (((0,), (0,)), ((), ()))
    sums_ref[...] += lax.dot_general(og, t, cdims,
                                     preferred_element_type=jnp.float32,
                                     precision=lax.Precision.HIGHEST)
    cnts_ref[...] += lax.dot_general(og, jnp.ones((BN, HID), jnp.float32),
                                     cdims, preferred_element_type=jnp.float32,
                                     precision=lax.Precision.HIGHEST)

    @pl.when(i == pl.num_programs(0) - 1)
    def _():
        hg = sums_ref[...] / jnp.maximum(cnts_ref[...], 1.0)
        z = _silu(_dot(hg, wp0[...]) + bp0[...])
        out_ref[...] = (jnp.sum(z * wp1[...], axis=1, keepdims=True)
                        + bp1[...])


def _node_init_call(na, x16, wemb, bemb, wi, bi, wj):
    grid = (N // BN,)
    return pl.pallas_call(
        _node_init_body,
        grid=grid,
        in_specs=[_rows((BN, HID)), _rows((BN, XW)),
                  _full((HID, HID)), _full((1, HID)),
                  _full((HID, HID)), _full((1, HID)), _full((HID, HID))],
        out_specs=[_rows((BN, HID)), _rows((BN, C)), _rows((BN, C))],
        out_shape=[jax.ShapeDtypeStruct((N, HID), jnp.float32),
                   jax.ShapeDtypeStruct((N, C), jnp.float32),
                   jax.ShapeDtypeStruct((N, C), jnp.float32)],
    )(na, x16, wemb, bemb, wi, bi, wj)


def _edge_call(u1, u2, ea, w):
    grid = (E // BE,)
    return pl.pallas_call(
        _edge_body,
        grid=grid,
        in_specs=[_rows((BE, C)), _rows((BE, C)), _rows((BE, EDIM)),
                  _full((EDIM, HID)), _full((1, HID)),
                  _full((HID, HID)), _full((1, HID)),
                  _full((1, HID)), _full((1, 1)),
                  _full((HID, HID)), _full((1, HID)),
                  _full((1, HID)), _full((1, 1))],
        out_specs=[_rows((BE, C))],
        out_shape=[jax.ShapeDtypeStruct((E, C), jnp.float32)],
    )(u1, u2, ea, w["we"], w["wr"], w["we1"], w["be1"], w["winf"], w["binf"],
      w["wx0"], w["bx0"], w["wx1"], w["bx1"])[0]


def _node_mid_call(h, x16, p0, p1, w, wi, bi, wj):
    grid = (N // BN,)
    return pl.pallas_call(
        _node_mid_body,
        grid=grid,
        in_specs=[_rows((BN, HID)), _rows((BN, XW)),
                  _rows((BN, C)), _rows((BN, C)),
                  _full((HID, HID)), _full((HID, HID)), _full((1, HID)),
                  _full((HID, HID)), _full((1, HID)),
                  _full((HID, HID)), _full((1, HID)), _full((HID, HID))],
        out_specs=[_rows((BN, HID)), _rows((BN, XW)),
                   _rows((BN, C)), _rows((BN, C))],
        out_shape=[jax.ShapeDtypeStruct((N, HID), jnp.float32),
                   jax.ShapeDtypeStruct((N, XW), jnp.float32),
                   jax.ShapeDtypeStruct((N, C), jnp.float32),
                   jax.ShapeDtypeStruct((N, C), jnp.float32)],
    )(h, x16, p0, p1, w["wh0h"], w["wh0m"], w["bh0"], w["wh1"], w["bh1"],
      wi, bi, wj)


def _node_last_call(h, p0, p1, w):
    grid = (N // BN,)
    return pl.pallas_call(
        _node_last_body,
        grid=grid,
        in_specs=[_rows((BN, HID)), _rows((BN, C)), _rows((BN, C)),
                  _full((HID, HID)), _full((HID, HID)), _full((1, HID)),
                  _full((HID, HID)), _full((1, HID))],
        out_specs=[_rows((BN, HID))],
        out_shape=[jax.ShapeDtypeStruct((N, HID), jnp.float32)],
    )(h, p0, p1, w["wh0h"], w["wh0m"], w["bh0"], w["wh1"], w["bh1"])[0]


def _readout_call(h, bids, w):
    grid = (N // BN,)
    return pl.pallas_call(
        _readout_body,
        grid=grid,
        in_specs=[_rows((BN, HID)), _rows((BN, 1)),
                  _full((HID, HID)), _full((1, HID)),
                  _full((HID, HID)), _full((1, HID)),
                  _full((HID, HID)), _full((1, HID)),
                  _full((1, HID)), _full((1, 1))],
        out_specs=[_full((NG, HID)), _full((NG, HID)), _full((NG, 1))],
        out_shape=[jax.ShapeDtypeStruct((NG, HID), jnp.float32),
                   jax.ShapeDtypeStruct((NG, HID), jnp.float32),
                   jax.ShapeDtypeStruct((NG, 1), jnp.float32)],
    )(h, bids, w["w0"], w["b0"], w["w1"], w["b1"],
      w["wp0"], w["bp0"], w["wp1"], w["bp1"])[2]


def _row(v):
    return v.reshape(1, -1)


def kernel(node_attr, pos, edge_attr, params, edge_index, batch):
    src = edge_index[0]
    dst = edge_index[1]
    x16 = jnp.pad(pos, ((0, 0), (0, XW - 3)))
    dst3 = dst.reshape(NTILES, NCH, CHUNK)
    src3 = src.reshape(NTILES, NCH, CHUNK)
    zeros = jnp.zeros((N, C), jnp.float32)
    bids = batch.reshape(N, 1)

    layers = params["layers"]

    def e0_split(lp):
        w = lp["e0"]["w"]
        return (w[:HID], _row(lp["e0"]["b"]), w[HID:2 * HID],
                w[2 * HID:2 * HID + 1], w[2 * HID + 1:])

    def edge_w(lp):
        _, _, _, wr, we = e0_split(lp)
        return {
            "we": we, "wr": wr,
            "we1": lp["e1"]["w"], "be1": _row(lp["e1"]["b"]),
            "winf": _row(lp["inf"]["w"][:, 0]),
            "binf": lp["inf"]["b"].reshape(1, 1),
            "wx0": lp["x0"]["w"], "bx0": _row(lp["x0"]["b"]),
            "wx1": _row(lp["x1"]["w"][:, 0]),
            "bx1": lp["x1"]["b"].reshape(1, 1),
        }

    def node_w(lp):
        return {
            "wh0h": lp["h0"]["w"][:HID], "wh0m": lp["h0"]["w"][HID:],
            "bh0": _row(lp["h0"]["b"]),
            "wh1": lp["h1"]["w"], "bh1": _row(lp["h1"]["b"]),
        }

    wi0, bi0, wj0, _, _ = e0_split(layers[0])
    h, t1, t2 = _node_init_call(
        node_attr, x16, params["embedding"]["w"],
        _row(params["embedding"]["b"]), wi0, bi0, wj0)

    sc_gather, sc_scatter = _sc_kernels()
    for l in range(len(layers)):
        lp = layers[l]
        u1, u2 = sc_gather(t1, t2, dst3, src3)
        v = _edge_call(u1, u2, edge_attr, edge_w(lp))
        p = sc_scatter(v, dst3, zeros).reshape(2, N, C)
        if l + 1 < len(layers):
            win, bin_, wjn, _, _ = e0_split(layers[l + 1])
            h, x16, t1, t2 = _node_mid_call(h, x16, p[0], p[1],
                                            node_w(lp), win, bin_, wjn)
        else:
            h = _node_last_call(h, p[0], p[1], node_w(lp))

    rw = {
        "w0": params["lin0"]["w"], "b0": _row(params["lin0"]["b"]),
        "w1": params["lin1"]["w"], "b1": _row(params["lin1"]["b"]),
        "wp0": params["pred0"]["w"], "bp0": _row(params["pred0"]["b"]),
        "wp1": _row(params["pred1"]["w"][:, 0]),
        "bp1": params["pred1"]["b"].reshape(1, 1),
    }
    out = _readout_call(h, bids, rw)
    return out.reshape(-1)


# speedup vs baseline: 3.0508x; 1.5618x over previous
import functools

import jax
import jax.numpy as jnp
from jax import lax
from jax.experimental import pallas as pl
from jax.experimental.pallas import tpu as pltpu
from jax.experimental.pallas import tpu_sc as plsc

N = 10000
E = 320000
HID = 128
EDIM = 16
NG = 64
XW = 16

NTILES = 32
EPT = E // NTILES
CHUNK = 80
NCH = EPT // CHUNK
NROW = N // 16
WTILES = 10
WROW = N // WTILES

BE = 2000
BN = 2000


def _sigmoid(x):
    return 0.5 * jnp.tanh(0.5 * x) + 0.5


def _silu(x):
    return x * _sigmoid(x)


def _gather2_body(w, t1_hbm, t2_hbm, dst3_hbm, src3_hbm, u1_hbm, u2_hbm,
                  idxd_v, idxs_v, u1_v, u2_v, sem1, sem2):
    wid = lax.axis_index("s") * 2 + lax.axis_index("c")
    pltpu.sync_copy(dst3_hbm.at[wid], idxd_v)
    pltpu.sync_copy(src3_hbm.at[wid], idxs_v)

    def body(k, carry):
        base = wid * EPT + k * CHUNK
        cp1 = pltpu.make_async_copy(t1_hbm.at[idxd_v.at[k]], u1_v, sem1)
        cp2 = pltpu.make_async_copy(t2_hbm.at[idxs_v.at[k]], u2_v, sem2)
        cp1.start()
        cp2.start()
        cp1.wait()
        cp2.wait()
        pltpu.sync_copy(u1_v, u1_hbm.at[pl.ds(base, CHUNK)])
        pltpu.sync_copy(u2_v, u2_hbm.at[pl.ds(base, CHUNK)])
        return carry

    lax.fori_loop(0, NCH, body, 0)


def _scatter_body(w, nw, wrow, v_hbm, dst3_hbm, zeros_hbm, p_hbm,
                  acc_sh, idx_v, v_v):
    c = lax.axis_index("c")
    s = lax.axis_index("s")
    wid = s * 2 + c
    @pl.when(s < nw)
    def _():
        pltpu.sync_copy(zeros_hbm.at[pl.ds(s * wrow, wrow)],
                        acc_sh.at[pl.ds(s * wrow, wrow)])
    plsc.subcore_barrier()
    pltpu.sync_copy(dst3_hbm.at[wid], idx_v)

    def body(k, carry):
        base = wid * EPT + k * CHUNK
        pltpu.sync_copy(v_hbm.at[pl.ds(base, CHUNK)], v_v)
        pltpu.sync_copy(v_v, acc_sh.at[idx_v.at[k]], add=True)
        return carry

    lax.fori_loop(0, NCH, body, 0)
    plsc.subcore_barrier()
    @pl.when(s < nw)
    def _():
        pltpu.sync_copy(acc_sh.at[pl.ds(s * wrow, wrow)], p_hbm.at[c, s])


@functools.cache
def _sc_kernels():
    mesh = plsc.VectorSubcoreMesh(core_axis_name="c", subcore_axis_name="s")
    sc_tiling = pltpu.CompilerParams(use_tc_tiling_on_sc=False)

    def gather2(width, params):
        return pl.kernel(
            functools.partial(_gather2_body, width),
            out_type=[jax.ShapeDtypeStruct((E, width), jnp.float32),
                      jax.ShapeDtypeStruct((E, width), jnp.float32)],
            mesh=mesh,
            compiler_params=params,
            scratch_types=[pltpu.VMEM((NCH, CHUNK), jnp.int32),
                           pltpu.VMEM((NCH, CHUNK), jnp.int32),
                           pltpu.VMEM((CHUNK, width), jnp.float32),
                           pltpu.VMEM((CHUNK, width), jnp.float32),
                           pltpu.SemaphoreType.DMA,
                           pltpu.SemaphoreType.DMA],
        )

    def scatter(width, nw, wrow, params):
        return pl.kernel(
            functools.partial(_scatter_body, width, nw, wrow),
            out_type=jax.ShapeDtypeStruct((2, nw, wrow, width), jnp.float32),
            mesh=mesh,
            compiler_params=params,
            scratch_types=[pltpu.VMEM_SHARED((N, width), jnp.float32),
                           pltpu.VMEM((NCH, CHUNK), jnp.int32),
                           pltpu.VMEM((CHUNK, width), jnp.float32)],
        )

    return {
        "gather_f": gather2(HID, None),
        "gather_x": gather2(XW, sc_tiling),
        "scatter_m": scatter(HID, WTILES, WROW, None),
        "scatter_x": scatter(XW, 16, NROW, sc_tiling),
    }


def _full(shape):
    return pl.BlockSpec(shape, lambda i: (0, 0))


def _rows(shape):
    return pl.BlockSpec(shape, lambda i: (i, 0))


def _dot(a, b):
    return jnp.dot(a, b, preferred_element_type=jnp.float32)


def _b(x):
    return x.astype(jnp.bfloat16).astype(jnp.float32)


def _node_init_body(na_ref, wemb, bemb, wi, bi, wj, h_ref, t1_ref, t2_ref):
    h = _dot(na_ref[...], wemb[...]) + bemb[...]
    h_ref[...] = h
    t1_ref[...] = _dot(h, wi[...]) + bi[...]
    t2_ref[...] = _dot(h, wj[...])


def _edge_body(u1_ref, u2_ref, x1_ref, x2_ref, ea_ref, we, wr, we1, be1,
               winf, binf, wx0, bx0, wx1, bx1, m_ref, dx_ref):
    g = u1_ref[...] + u2_ref[...]
    d = x1_ref[...] - x2_ref[...]
    r2 = jnp.sum(d * d, axis=1, keepdims=True)
    pre = g + _b(r2) * _b(wr[...]) + _dot(ea_ref[...], we[...])
    u = _silu(pre)
    m1 = _silu(_dot(u, we1[...]) + be1[...])
    gate = _sigmoid(
        jnp.sum(_b(m1) * _b(winf[...]), axis=1, keepdims=True) + binf[...])
    m = gate * m1
    t = _silu(_dot(m, wx0[...]) + bx0[...])
    coef = jnp.sum(_b(t) * _b(wx1[...]), axis=1, keepdims=True) + bx1[...]
    m_ref[...] = m
    dx_ref[...] = d * coef


def _node_mid_body(h_ref, x_ref, p0_ref, p1_ref, q0_ref, q1_ref,
                   wh0h, wh0m, bh0, wh1, bh1,
                   wi, bi, wj, hn_ref, xn_ref, t1_ref, t2_ref):
    h = h_ref[...]
    magg = p0_ref[...] + p1_ref[...]
    xn_ref[...] = x_ref[...] + q0_ref[...] + q1_ref[...]
    u = _silu(_dot(h, wh0h[...]) + _dot(magg, wh0m[...]) + bh0[...])
    hn = _dot(u, wh1[...]) + bh1[...]
    hn_ref[...] = hn
    t1_ref[...] = _dot(hn, wi[...]) + bi[...]
    t2_ref[...] = _dot(hn, wj[...])


def _node_last_body(h_ref, p0_ref, p1_ref, wh0h, wh0m, bh0, wh1, bh1,
                    hn_ref):
    h = h_ref[...]
    magg = p0_ref[...] + p1_ref[...]
    u = _silu(_dot(h, wh0h[...]) + _dot(magg, wh0m[...]) + bh0[...])
    hn_ref[...] = _dot(u, wh1[...]) + bh1[...]


def _readout_body(h_ref, b_ref, w0, b0, w1, b1, wp0, bp0, wp1, bp1,
                  sums_ref, cnts_ref, out_ref):
    i = pl.program_id(0)

    @pl.when(i == 0)
    def _():
        sums_ref[...] = jnp.zeros_like(sums_ref)
        cnts_ref[...] = jnp.zeros_like(cnts_ref)
        out_ref[...] = jnp.zeros_like(out_ref)

    t = _silu(_dot(h_ref[...], w0[...]) + b0[...])
    t = _dot(t, w1[...]) + b1[...]
    og = (b_ref[...] == lax.broadcasted_iota(jnp.int32, (BN, NG), 1)
          ).astype(jnp.float32)
    cdims = (((0,), (0,)), ((), ()))
    sums_ref[...] += lax.dot_general(og, t, cdims,
                                     preferred_element_type=jnp.float32,
                                     precision=lax.Precision.HIGHEST)
    cnts_ref[...] += lax.dot_general(og, jnp.ones((BN, HID), jnp.float32),
                                     cdims, preferred_element_type=jnp.float32,
                                     precision=lax.Precision.HIGHEST)

    @pl.when(i == pl.num_programs(0) - 1)
    def _():
        hg = sums_ref[...] / jnp.maximum(cnts_ref[...], 1.0)
        z = _silu(_dot(hg, wp0[...]) + bp0[...])
        out_ref[...] = (jnp.sum(z * wp1[...], axis=1, keepdims=True)
                        + bp1[...])


def _node_init_call(na, wemb, bemb, wi, bi, wj):
    grid = (N // BN,)
    return pl.pallas_call(
        _node_init_body,
        grid=grid,
        in_specs=[_rows((BN, HID)),
                  _full((HID, HID)), _full((1, HID)),
                  _full((HID, HID)), _full((1, HID)), _full((HID, HID))],
        out_specs=[_rows((BN, HID)), _rows((BN, HID)), _rows((BN, HID))],
        out_shape=[jax.ShapeDtypeStruct((N, HID), jnp.float32),
                   jax.ShapeDtypeStruct((N, HID), jnp.float32),
                   jax.ShapeDtypeStruct((N, HID), jnp.float32)],
    )(na, wemb, bemb, wi, bi, wj)


def _edge_call(u1, u2, x1, x2, ea, w):
    grid = (E // BE,)
    return pl.pallas_call(
        _edge_body,
        grid=grid,
        in_specs=[_rows((BE, HID)), _rows((BE, HID)),
                  _rows((BE, XW)), _rows((BE, XW)), _rows((BE, EDIM)),
                  _full((EDIM, HID)), _full((1, HID)),
                  _full((HID, HID)), _full((1, HID)),
                  _full((1, HID)), _full((1, 1)),
                  _full((HID, HID)), _full((1, HID)),
                  _full((1, HID)), _full((1, 1))],
        out_specs=[_rows((BE, HID)), _rows((BE, XW))],
        out_shape=[jax.ShapeDtypeStruct((E, HID), jnp.float32),
                   jax.ShapeDtypeStruct((E, XW), jnp.float32)],
    )(u1, u2, x1, x2, ea, w["we"], w["wr"], w["we1"], w["be1"], w["winf"],
      w["binf"], w["wx0"], w["bx0"], w["wx1"], w["bx1"])


def _node_mid_call(h, x16, p0, p1, q0, q1, w, wi, bi, wj):
    grid = (N // BN,)
    return pl.pallas_call(
        _node_mid_body,
        grid=grid,
        in_specs=[_rows((BN, HID)), _rows((BN, XW)),
                  _rows((BN, HID)), _rows((BN, HID)),
                  _rows((BN, XW)), _rows((BN, XW)),
                  _full((HID, HID)), _full((HID, HID)), _full((1, HID)),
                  _full((HID, HID)), _full((1, HID)),
                  _full((HID, HID)), _full((1, HID)), _full((HID, HID))],
        out_specs=[_rows((BN, HID)), _rows((BN, XW)),
                   _rows((BN, HID)), _rows((BN, HID))],
        out_shape=[jax.ShapeDtypeStruct((N, HID), jnp.float32),
                   jax.ShapeDtypeStruct((N, XW), jnp.float32),
                   jax.ShapeDtypeStruct((N, HID), jnp.float32),
                   jax.ShapeDtypeStruct((N, HID), jnp.float32)],
    )(h, x16, p0, p1, q0, q1, w["wh0h"], w["wh0m"], w["bh0"], w["wh1"],
      w["bh1"], wi, bi, wj)


def _node_last_call(h, p0, p1, w):
    grid = (N // BN,)
    return pl.pallas_call(
        _node_last_body,
        grid=grid,
        in_specs=[_rows((BN, HID)), _rows((BN, HID)), _rows((BN, HID)),
                  _full((HID, HID)), _full((HID, HID)), _full((1, HID)),
                  _full((HID, HID)), _full((1, HID))],
        out_specs=[_rows((BN, HID))],
        out_shape=[jax.ShapeDtypeStruct((N, HID), jnp.float32)],
    )(h, p0, p1, w["wh0h"], w["wh0m"], w["bh0"], w["wh1"], w["bh1"])[0]


def _readout_call(h, bids, w):
    grid = (N // BN,)
    return pl.pallas_call(
        _readout_body,
        grid=grid,
        in_specs=[_rows((BN, HID)), _rows((BN, 1)),
                  _full((HID, HID)), _full((1, HID)),
                  _full((HID, HID)), _full((1, HID)),
                  _full((HID, HID)), _full((1, HID)),
                  _full((1, HID)), _full((1, 1))],
        out_specs=[_full((NG, HID)), _full((NG, HID)), _full((NG, 1))],
        out_shape=[jax.ShapeDtypeStruct((NG, HID), jnp.float32),
                   jax.ShapeDtypeStruct((NG, HID), jnp.float32),
                   jax.ShapeDtypeStruct((NG, 1), jnp.float32)],
    )(h, bids, w["w0"], w["b0"], w["w1"], w["b1"],
      w["wp0"], w["bp0"], w["wp1"], w["bp1"])[2]


def _row(v):
    return v.reshape(1, -1)


def kernel(node_attr, pos, edge_attr, params, edge_index, batch):
    src = edge_index[0]
    dst = edge_index[1]
    x16 = jnp.pad(pos, ((0, 0), (0, XW - 3)))
    dst3 = dst.reshape(NTILES, NCH, CHUNK)
    src3 = src.reshape(NTILES, NCH, CHUNK)
    zeros_m = jnp.zeros((N, HID), jnp.float32)
    zeros_x = jnp.zeros((N, XW), jnp.float32)
    bids = batch.reshape(N, 1)

    layers = params["layers"]

    def e0_split(lp):
        w = lp["e0"]["w"]
        return (w[:HID], _row(lp["e0"]["b"]), w[HID:2 * HID],
                w[2 * HID:2 * HID + 1], w[2 * HID + 1:])

    def edge_w(lp):
        _, _, _, wr, we = e0_split(lp)
        return {
            "we": we, "wr": wr,
            "we1": lp["e1"]["w"], "be1": _row(lp["e1"]["b"]),
            "winf": _row(lp["inf"]["w"][:, 0]),
            "binf": lp["inf"]["b"].reshape(1, 1),
            "wx0": lp["x0"]["w"], "bx0": _row(lp["x0"]["b"]),
            "wx1": _row(lp["x1"]["w"][:, 0]),
            "bx1": lp["x1"]["b"].reshape(1, 1),
        }

    def node_w(lp):
        return {
            "wh0h": lp["h0"]["w"][:HID], "wh0m": lp["h0"]["w"][HID:],
            "bh0": _row(lp["h0"]["b"]),
            "wh1": lp["h1"]["w"], "bh1": _row(lp["h1"]["b"]),
        }

    wi0, bi0, wj0, _, _ = e0_split(layers[0])
    h, t1, t2 = _node_init_call(node_attr, params["embedding"]["w"],
                                _row(params["embedding"]["b"]), wi0, bi0, wj0)

    sc = _sc_kernels()
    for l in range(len(layers)):
        lp = layers[l]
        u1, u2 = sc["gather_f"](t1, t2, dst3, src3)
        xg1, xg2 = sc["gather_x"](x16, x16, dst3, src3)
        m, dx = _edge_call(u1, u2, xg1, xg2, edge_attr, edge_w(lp))
        p = sc["scatter_m"](m, dst3, zeros_m).reshape(2, N, HID)
        q = sc["scatter_x"](dx, dst3, zeros_x).reshape(2, N, XW)
        if l + 1 < len(layers):
            win, bin_, wjn, _, _ = e0_split(layers[l + 1])
            h, x16, t1, t2 = _node_mid_call(h, x16, p[0], p[1], q[0], q[1],
                                            node_w(lp), win, bin_, wjn)
        else:
            h = _node_last_call(h, p[0], p[1], node_w(lp))

    rw = {
        "w0": params["lin0"]["w"], "b0": _row(params["lin0"]["b"]),
        "w1": params["lin1"]["w"], "b1": _row(params["lin1"]["b"]),
        "wp0": params["pred0"]["w"], "bp0": _row(params["pred0"]["b"]),
        "wp1": _row(params["pred1"]["w"][:, 0]),
        "bp1": params["pred1"]["b"].reshape(1, 1),
    }
    out = _readout_call(h, bids, rw)
    return out.reshape(-1)


# speedup vs baseline: 3.5927x; 1.1776x over previous
import functools

import jax
import jax.numpy as jnp
from jax import lax
from jax.experimental import pallas as pl
from jax.experimental.pallas import tpu as pltpu
from jax.experimental.pallas import tpu_sc as plsc

N = 10000
E = 320000
HID = 128
EDIM = 16
NG = 64
XW = 16

NTILES = 32
EPT = E // NTILES
CHUNK = 80
NCH = EPT // CHUNK
NROW = N // 16
WTILES = 10
WROW = N // WTILES

BE = 4000
BN = 2000


def _sigmoid(x):
    return 0.5 * jnp.tanh(0.5 * x) + 0.5


def _silu(x):
    return x * _sigmoid(x)


def _gather_comb_body(w, sub, t1_hbm, t2_hbm, dst3_hbm, src3_hbm, out_hbm,
                      idxd_v, idxs_v, b1a, b2a, b1b, b2b, s1a, s2a, s1b, s2b):
    wid = lax.axis_index("s") * 2 + lax.axis_index("c")
    pltpu.sync_copy(dst3_hbm.at[wid], idxd_v)
    pltpu.sync_copy(src3_hbm.at[wid], idxs_v)
    nsl = w // 16

    def start(k, b1, b2, s1, s2):
        pltpu.make_async_copy(t1_hbm.at[idxd_v.at[k]], b1, s1).start()
        pltpu.make_async_copy(t2_hbm.at[idxs_v.at[k]], b2, s2).start()

    def finish(k, b1, b2, s1, s2):
        pltpu.make_async_copy(t1_hbm.at[idxd_v.at[k]], b1, s1).wait()
        pltpu.make_async_copy(t2_hbm.at[idxs_v.at[k]], b2, s2).wait()

        def vrow(j, carry):
            for q in range(16):
                r = j * 16 + q
                for t in range(nsl):
                    sl = pl.ds(t * 16, 16)
                    if sub:
                        b1[r, sl] = b1[r, sl] - b2[r, sl]
                    else:
                        b1[r, sl] = b1[r, sl] + b2[r, sl]
            return carry

        lax.fori_loop(0, CHUNK // 16, vrow, 0)
        pltpu.sync_copy(b1, out_hbm.at[pl.ds(wid * EPT + k * CHUNK, CHUNK)])

    start(0, b1a, b2a, s1a, s2a)

    def body(i, carry):
        start(2 * i + 1, b1b, b2b, s1b, s2b)
        finish(2 * i, b1a, b2a, s1a, s2a)
        start(2 * i + 2, b1a, b2a, s1a, s2a)
        finish(2 * i + 1, b1b, b2b, s1b, s2b)
        return carry

    lax.fori_loop(0, (NCH - 1) // 2, body, 0)
    finish(NCH - 1, b1a, b2a, s1a, s2a)


def _scatter_body(w, nw, wrow, v_hbm, dst3_hbm, zeros_hbm, p_hbm,
                  acc_sh, idx_v, v_v):
    c = lax.axis_index("c")
    s = lax.axis_index("s")
    wid = s * 2 + c
    @pl.when(s < nw)
    def _():
        pltpu.sync_copy(zeros_hbm.at[pl.ds(s * wrow, wrow)],
                        acc_sh.at[pl.ds(s * wrow, wrow)])
    plsc.subcore_barrier()
    pltpu.sync_copy(dst3_hbm.at[wid], idx_v)

    def body(k, carry):
        base = wid * EPT + k * CHUNK
        pltpu.sync_copy(v_hbm.at[pl.ds(base, CHUNK)], v_v)
        pltpu.sync_copy(v_v, acc_sh.at[idx_v.at[k]], add=True)
        return carry

    lax.fori_loop(0, NCH, body, 0)
    plsc.subcore_barrier()
    @pl.when(s < nw)
    def _():
        pltpu.sync_copy(acc_sh.at[pl.ds(s * wrow, wrow)], p_hbm.at[c, s])


@functools.cache
def _sc_kernels():
    mesh = plsc.VectorSubcoreMesh(core_axis_name="c", subcore_axis_name="s")
    sc_tiling = pltpu.CompilerParams(use_tc_tiling_on_sc=False)

    def gather_comb(width, sub, params):
        return pl.kernel(
            functools.partial(_gather_comb_body, width, sub),
            out_type=jax.ShapeDtypeStruct((E, width), jnp.float32),
            mesh=mesh,
            compiler_params=params,
            scratch_types=[pltpu.VMEM((NCH, CHUNK), jnp.int32),
                           pltpu.VMEM((NCH, CHUNK), jnp.int32),
                           pltpu.VMEM((CHUNK, width), jnp.float32),
                           pltpu.VMEM((CHUNK, width), jnp.float32),
                           pltpu.VMEM((CHUNK, width), jnp.float32),
                           pltpu.VMEM((CHUNK, width), jnp.float32),
                           pltpu.SemaphoreType.DMA,
                           pltpu.SemaphoreType.DMA,
                           pltpu.SemaphoreType.DMA,
                           pltpu.SemaphoreType.DMA],
        )

    def scatter(width, nw, wrow, params):
        return pl.kernel(
            functools.partial(_scatter_body, width, nw, wrow),
            out_type=jax.ShapeDtypeStruct((2, nw, wrow, width), jnp.float32),
            mesh=mesh,
            compiler_params=params,
            scratch_types=[pltpu.VMEM_SHARED((N, width), jnp.float32),
                           pltpu.VMEM((NCH, CHUNK), jnp.int32),
                           pltpu.VMEM((CHUNK, width), jnp.float32)],
        )

    return {
        "gather_f": gather_comb(HID, False, None),
        "gather_x": gather_comb(XW, True, sc_tiling),
        "scatter_m": scatter(HID, WTILES, WROW, None),
        "scatter_x": scatter(XW, 16, NROW, sc_tiling),
    }


def _full(shape):
    return pl.BlockSpec(shape, lambda i: (0, 0))


def _rows(shape):
    return pl.BlockSpec(shape, lambda i: (i, 0))


def _dot(a, b):
    return jnp.dot(a, b, preferred_element_type=jnp.float32)


def _b(x):
    return x.astype(jnp.bfloat16).astype(jnp.float32)


def _node_init_body(na_ref, wemb, bemb, wi, bi, wj, h_ref, t1_ref, t2_ref):
    h = _dot(na_ref[...], wemb[...]) + bemb[...]
    h_ref[...] = h
    t1_ref[...] = _dot(h, wi[...]) + bi[...]
    t2_ref[...] = _dot(h, wj[...])


def _edge_body(g_ref, d_ref, ea_ref, we, wr, we1, be1,
               winf, binf, wx0, bx0, wx1, bx1, m_ref, dx_ref):
    g = g_ref[...]
    d = d_ref[...]
    r2 = jnp.sum(d * d, axis=1, keepdims=True)
    pre = g + _b(r2) * _b(wr[...]) + _dot(ea_ref[...], we[...])
    u = _silu(pre)
    m1 = _silu(_dot(u, we1[...]) + be1[...])
    gate = _sigmoid(
        jnp.sum(_b(m1) * _b(winf[...]), axis=1, keepdims=True) + binf[...])
    m = gate * m1
    t = _silu(_dot(m, wx0[...]) + bx0[...])
    coef = jnp.sum(_b(t) * _b(wx1[...]), axis=1, keepdims=True) + bx1[...]
    m_ref[...] = m
    dx_ref[...] = d * coef


def _node_mid_body(h_ref, x_ref, p0_ref, p1_ref, q0_ref, q1_ref,
                   wh0h, wh0m, bh0, wh1, bh1,
                   wi, bi, wj, hn_ref, xn_ref, t1_ref, t2_ref):
    h = h_ref[...]
    magg = p0_ref[...] + p1_ref[...]
    xn_ref[...] = x_ref[...] + q0_ref[...] + q1_ref[...]
    u = _silu(_dot(h, wh0h[...]) + _dot(magg, wh0m[...]) + bh0[...])
    hn = _dot(u, wh1[...]) + bh1[...]
    hn_ref[...] = hn
    t1_ref[...] = _dot(hn, wi[...]) + bi[...]
    t2_ref[...] = _dot(hn, wj[...])


def _node_last_body(h_ref, p0_ref, p1_ref, wh0h, wh0m, bh0, wh1, bh1,
                    hn_ref):
    h = h_ref[...]
    magg = p0_ref[...] + p1_ref[...]
    u = _silu(_dot(h, wh0h[...]) + _dot(magg, wh0m[...]) + bh0[...])
    hn_ref[...] = _dot(u, wh1[...]) + bh1[...]


def _readout_body(h_ref, b_ref, w0, b0, w1, b1, wp0, bp0, wp1, bp1,
                  sums_ref, cnts_ref, out_ref):
    i = pl.program_id(0)

    @pl.when(i == 0)
    def _():
        sums_ref[...] = jnp.zeros_like(sums_ref)
        cnts_ref[...] = jnp.zeros_like(cnts_ref)
        out_ref[...] = jnp.zeros_like(out_ref)

    t = _silu(_dot(h_ref[...], w0[...]) + b0[...])
    t = _dot(t, w1[...]) + b1[...]
    og = (b_ref[...] == lax.broadcasted_iota(jnp.int32, (BN, NG), 1)
          ).astype(jnp.float32)
    cdims = (((0,), (0,)), ((), ()))
    sums_ref[...] += lax.dot_general(og, t, cdims,
                                     preferred_element_type=jnp.float32,
                                     precision=lax.Precision.HIGHEST)
    cnts_ref[...] += lax.dot_general(og, jnp.ones((BN, HID), jnp.float32),
                                     cdims, preferred_element_type=jnp.float32,
                                     precision=lax.Precision.HIGHEST)

    @pl.when(i == pl.num_programs(0) - 1)
    def _():
        hg = sums_ref[...] / jnp.maximum(cnts_ref[...], 1.0)
        z = _silu(_dot(hg, wp0[...]) + bp0[...])
        out_ref[...] = (jnp.sum(z * wp1[...], axis=1, keepdims=True)
                        + bp1[...])


def _node_init_call(na, wemb, bemb, wi, bi, wj):
    grid = (N // BN,)
    return pl.pallas_call(
        _node_init_body,
        grid=grid,
        in_specs=[_rows((BN, HID)),
                  _full((HID, HID)), _full((1, HID)),
                  _full((HID, HID)), _full((1, HID)), _full((HID, HID))],
        out_specs=[_rows((BN, HID)), _rows((BN, HID)), _rows((BN, HID))],
        out_shape=[jax.ShapeDtypeStruct((N, HID), jnp.float32),
                   jax.ShapeDtypeStruct((N, HID), jnp.float32),
                   jax.ShapeDtypeStruct((N, HID), jnp.float32)],
    )(na, wemb, bemb, wi, bi, wj)


def _edge_call(g, d, ea, w):
    grid = (E // BE,)
    return pl.pallas_call(
        _edge_body,
        grid=grid,
        in_specs=[_rows((BE, HID)), _rows((BE, XW)), _rows((BE, EDIM)),
                  _full((EDIM, HID)), _full((1, HID)),
                  _full((HID, HID)), _full((1, HID)),
                  _full((1, HID)), _full((1, 1)),
                  _full((HID, HID)), _full((1, HID)),
                  _full((1, HID)), _full((1, 1))],
        out_specs=[_rows((BE, HID)), _rows((BE, XW))],
        out_shape=[jax.ShapeDtypeStruct((E, HID), jnp.float32),
                   jax.ShapeDtypeStruct((E, XW), jnp.float32)],
    )(g, d, ea, w["we"], w["wr"], w["we1"], w["be1"], w["winf"],
      w["binf"], w["wx0"], w["bx0"], w["wx1"], w["bx1"])


def _node_mid_call(h, x16, p0, p1, q0, q1, w, wi, bi, wj):
    grid = (N // BN,)
    return pl.pallas_call(
        _node_mid_body,
        grid=grid,
        in_specs=[_rows((BN, HID)), _rows((BN, XW)),
                  _rows((BN, HID)), _rows((BN, HID)),
                  _rows((BN, XW)), _rows((BN, XW)),
                  _full((HID, HID)), _full((HID, HID)), _full((1, HID)),
                  _full((HID, HID)), _full((1, HID)),
                  _full((HID, HID)), _full((1, HID)), _full((HID, HID))],
        out_specs=[_rows((BN, HID)), _rows((BN, XW)),
                   _rows((BN, HID)), _rows((BN, HID))],
        out_shape=[jax.ShapeDtypeStruct((N, HID), jnp.float32),
                   jax.ShapeDtypeStruct((N, XW), jnp.float32),
                   jax.ShapeDtypeStruct((N, HID), jnp.float32),
                   jax.ShapeDtypeStruct((N, HID), jnp.float32)],
    )(h, x16, p0, p1, q0, q1, w["wh0h"], w["wh0m"], w["bh0"], w["wh1"],
      w["bh1"], wi, bi, wj)


def _node_last_call(h, p0, p1, w):
    grid = (N // BN,)
    return pl.pallas_call(
        _node_last_body,
        grid=grid,
        in_specs=[_rows((BN, HID)), _rows((BN, HID)), _rows((BN, HID)),
                  _full((HID, HID)), _full((HID, HID)), _full((1, HID)),
                  _full((HID, HID)), _full((1, HID))],
        out_specs=[_rows((BN, HID))],
        out_shape=[jax.ShapeDtypeStruct((N, HID), jnp.float32)],
    )(h, p0, p1, w["wh0h"], w["wh0m"], w["bh0"], w["wh1"], w["bh1"])[0]


def _readout_call(h, bids, w):
    grid = (N // BN,)
    return pl.pallas_call(
        _readout_body,
        grid=grid,
        in_specs=[_rows((BN, HID)), _rows((BN, 1)),
                  _full((HID, HID)), _full((1, HID)),
                  _full((HID, HID)), _full((1, HID)),
                  _full((HID, HID)), _full((1, HID)),
                  _full((1, HID)), _full((1, 1))],
        out_specs=[_full((NG, HID)), _full((NG, HID)), _full((NG, 1))],
        out_shape=[jax.ShapeDtypeStruct((NG, HID), jnp.float32),
                   jax.ShapeDtypeStruct((NG, HID), jnp.float32),
                   jax.ShapeDtypeStruct((NG, 1), jnp.float32)],
    )(h, bids, w["w0"], w["b0"], w["w1"], w["b1"],
      w["wp0"], w["bp0"], w["wp1"], w["bp1"])[2]


def _row(v):
    return v.reshape(1, -1)


def kernel(node_attr, pos, edge_attr, params, edge_index, batch):
    src = edge_index[0]
    dst = edge_index[1]
    x16 = jnp.pad(pos, ((0, 0), (0, XW - 3)))
    dst3 = dst.reshape(NTILES, NCH, CHUNK)
    src3 = src.reshape(NTILES, NCH, CHUNK)
    zeros_m = jnp.zeros((N, HID), jnp.float32)
    zeros_x = jnp.zeros((N, XW), jnp.float32)
    bids = batch.reshape(N, 1)

    layers = params["layers"]

    def e0_split(lp):
        w = lp["e0"]["w"]
        return (w[:HID], _row(lp["e0"]["b"]), w[HID:2 * HID],
                w[2 * HID:2 * HID + 1], w[2 * HID + 1:])

    def edge_w(lp):
        _, _, _, wr, we = e0_split(lp)
        return {
            "we": we, "wr": wr,
            "we1": lp["e1"]["w"], "be1": _row(lp["e1"]["b"]),
            "winf": _row(lp["inf"]["w"][:, 0]),
            "binf": lp["inf"]["b"].reshape(1, 1),
            "wx0": lp["x0"]["w"], "bx0": _row(lp["x0"]["b"]),
            "wx1": _row(lp["x1"]["w"][:, 0]),
            "bx1": lp["x1"]["b"].reshape(1, 1),
        }

    def node_w(lp):
        return {
            "wh0h": lp["h0"]["w"][:HID], "wh0m": lp["h0"]["w"][HID:],
            "bh0": _row(lp["h0"]["b"]),
            "wh1": lp["h1"]["w"], "bh1": _row(lp["h1"]["b"]),
        }

    wi0, bi0, wj0, _, _ = e0_split(layers[0])
    h, t1, t2 = _node_init_call(node_attr, params["embedding"]["w"],
                                _row(params["embedding"]["b"]), wi0, bi0, wj0)

    sc = _sc_kernels()
    for l in range(len(layers)):
        lp = layers[l]
        g = sc["gather_f"](t1, t2, dst3, src3)
        d = sc["gather_x"](x16, x16, dst3, src3)
        m, dx = _edge_call(g, d, edge_attr, edge_w(lp))
        p = sc["scatter_m"](m, dst3, zeros_m).reshape(2, N, HID)
        q = sc["scatter_x"](dx, dst3, zeros_x).reshape(2, N, XW)
        if l + 1 < len(layers):
            win, bin_, wjn, _, _ = e0_split(layers[l + 1])
            h, x16, t1, t2 = _node_mid_call(h, x16, p[0], p[1], q[0], q[1],
                                            node_w(lp), win, bin_, wjn)
        else:
            h = _node_last_call(h, p[0], p[1], node_w(lp))

    rw = {
        "w0": params["lin0"]["w"], "b0": _row(params["lin0"]["b"]),
        "w1": params["lin1"]["w"], "b1": _row(params["lin1"]["b"]),
        "wp0": params["pred0"]["w"], "bp0": _row(params["pred0"]["b"]),
        "wp1": _row(params["pred1"]["w"][:, 0]),
        "bp1": params["pred1"]["b"].reshape(1, 1),
    }
    out = _readout_call(h, bids, rw)
    return out.reshape(-1)


# speedup vs baseline: 3.8930x; 1.0836x over previous
import functools

import jax
import jax.numpy as jnp
from jax import lax
from jax.experimental import pallas as pl
from jax.experimental.pallas import tpu as pltpu
from jax.experimental.pallas import tpu_sc as plsc

N = 10000
E = 320000
HID = 128
EDIM = 16
NG = 64
XW = 16

NTILES = 32
EPT = E // NTILES
CHUNK = 80
NCH = EPT // CHUNK
NROW = N // 16
WTILES = 10
WROW = N // WTILES

BE = 4000
BN = 2000


def _sigmoid(x):
    return 0.5 * jnp.tanh(0.5 * x) + 0.5


def _silu(x):
    return x * _sigmoid(x)


def _gather_comb_body(w, sub, nch, t1_hbm, t2_hbm, dst3_hbm, src3_hbm,
                      out_hbm, idxd_v, idxs_v, b1a, b2a, b1b, b2b,
                      s1a, s2a, s1b, s2b):
    wid = lax.axis_index("s") * 2 + lax.axis_index("c")
    pltpu.sync_copy(dst3_hbm.at[wid], idxd_v)
    pltpu.sync_copy(src3_hbm.at[wid], idxs_v)
    nsl = w // 16

    def start(k, b1, b2, s1, s2):
        pltpu.make_async_copy(t1_hbm.at[idxd_v.at[k]], b1, s1).start()
        pltpu.make_async_copy(t2_hbm.at[idxs_v.at[k]], b2, s2).start()

    def finish(k, b1, b2, s1, s2):
        pltpu.make_async_copy(t1_hbm.at[idxd_v.at[k]], b1, s1).wait()
        pltpu.make_async_copy(t2_hbm.at[idxs_v.at[k]], b2, s2).wait()

        def vrow(j, carry):
            for q in range(16):
                r = j * 16 + q
                for t in range(nsl):
                    sl = pl.ds(t * 16, 16)
                    if sub:
                        b1[r, sl] = b1[r, sl] - b2[r, sl]
                    else:
                        b1[r, sl] = b1[r, sl] + b2[r, sl]
            return carry

        lax.fori_loop(0, CHUNK // 16, vrow, 0)
        pltpu.sync_copy(
            b1, out_hbm.at[pl.ds(wid * (nch * CHUNK) + k * CHUNK, CHUNK)])

    start(0, b1a, b2a, s1a, s2a)

    def body(i, carry):
        start(2 * i + 1, b1b, b2b, s1b, s2b)
        finish(2 * i, b1a, b2a, s1a, s2a)
        start(2 * i + 2, b1a, b2a, s1a, s2a)
        finish(2 * i + 1, b1b, b2b, s1b, s2b)
        return carry

    lax.fori_loop(0, (nch - 1) // 2, body, 0)
    if nch % 2 == 1:
        finish(nch - 1, b1a, b2a, s1a, s2a)
    else:
        start(nch - 1, b1b, b2b, s1b, s2b)
        finish(nch - 2, b1a, b2a, s1a, s2a)
        finish(nch - 1, b1b, b2b, s1b, s2b)


def _scatter_body(w, nw, wrow, nch, v_hbm, dst3_hbm, zeros_hbm, p_hbm,
                  acc_sh, idx_v, v_v):
    c = lax.axis_index("c")
    s = lax.axis_index("s")
    wid = s * 2 + c
    @pl.when(s < nw)
    def _():
        pltpu.sync_copy(zeros_hbm.at[pl.ds(s * wrow, wrow)],
                        acc_sh.at[pl.ds(s * wrow, wrow)])
    plsc.subcore_barrier()
    pltpu.sync_copy(dst3_hbm.at[wid], idx_v)

    def body(k, carry):
        base = wid * (nch * CHUNK) + k * CHUNK
        pltpu.sync_copy(v_hbm.at[pl.ds(base, CHUNK)], v_v)
        pltpu.sync_copy(v_v, acc_sh.at[idx_v.at[k]], add=True)
        return carry

    lax.fori_loop(0, nch, body, 0)
    plsc.subcore_barrier()
    @pl.when(s < nw)
    def _():
        pltpu.sync_copy(acc_sh.at[pl.ds(s * wrow, wrow)], p_hbm.at[c, s])


@functools.cache
def _sc_kernels(nch):
    mesh = plsc.VectorSubcoreMesh(core_axis_name="c", subcore_axis_name="s")
    sc_tiling = pltpu.CompilerParams(use_tc_tiling_on_sc=False)
    e_half = NTILES * nch * CHUNK

    def gather_comb(width, sub, params):
        return pl.kernel(
            functools.partial(_gather_comb_body, width, sub, nch),
            out_type=jax.ShapeDtypeStruct((e_half, width), jnp.float32),
            mesh=mesh,
            compiler_params=params,
            scratch_types=[pltpu.VMEM((nch, CHUNK), jnp.int32),
                           pltpu.VMEM((nch, CHUNK), jnp.int32),
                           pltpu.VMEM((CHUNK, width), jnp.float32),
                           pltpu.VMEM((CHUNK, width), jnp.float32),
                           pltpu.VMEM((CHUNK, width), jnp.float32),
                           pltpu.VMEM((CHUNK, width), jnp.float32),
                           pltpu.SemaphoreType.DMA,
                           pltpu.SemaphoreType.DMA,
                           pltpu.SemaphoreType.DMA,
                           pltpu.SemaphoreType.DMA],
        )

    def scatter(width, nw, wrow, params):
        return pl.kernel(
            functools.partial(_scatter_body, width, nw, wrow, nch),
            out_type=jax.ShapeDtypeStruct((2, nw, wrow, width), jnp.float32),
            mesh=mesh,
            compiler_params=params,
            scratch_types=[pltpu.VMEM_SHARED((N, width), jnp.float32),
                           pltpu.VMEM((nch, CHUNK), jnp.int32),
                           pltpu.VMEM((CHUNK, width), jnp.float32)],
        )

    return {
        "gather_f": gather_comb(HID, False, None),
        "gather_x": gather_comb(XW, True, sc_tiling),
        "scatter_m": scatter(HID, WTILES, WROW, None),
        "scatter_x": scatter(XW, 16, NROW, sc_tiling),
    }


def _full(shape):
    return pl.BlockSpec(shape, lambda i: (0, 0))


def _rows(shape):
    return pl.BlockSpec(shape, lambda i: (i, 0))


def _dot(a, b):
    return jnp.dot(a, b, preferred_element_type=jnp.float32)


def _b(x):
    return x.astype(jnp.bfloat16).astype(jnp.float32)


def _node_init_body(na_ref, wemb, bemb, wi, bi, wj, h_ref, t1_ref, t2_ref):
    h = _dot(na_ref[...], wemb[...]) + bemb[...]
    h_ref[...] = h
    t1_ref[...] = _dot(h, wi[...]) + bi[...]
    t2_ref[...] = _dot(h, wj[...])


def _edge_body(g_ref, d_ref, ea_ref, we, wr, we1, be1,
               winf, binf, wx0, bx0, wx1, bx1, m_ref, dx_ref):
    g = g_ref[...]
    d = d_ref[...]
    r2 = jnp.sum(d * d, axis=1, keepdims=True)
    pre = g + _b(r2) * _b(wr[...]) + _dot(ea_ref[...], we[...])
    u = _silu(pre)
    m1 = _silu(_dot(u, we1[...]) + be1[...])
    gate = _sigmoid(
        jnp.sum(_b(m1) * _b(winf[...]), axis=1, keepdims=True) + binf[...])
    m = gate * m1
    t = _silu(_dot(m, wx0[...]) + bx0[...])
    coef = jnp.sum(_b(t) * _b(wx1[...]), axis=1, keepdims=True) + bx1[...]
    m_ref[...] = m
    dx_ref[...] = d * coef


def _node_mid_body(h_ref, x_ref, p0_ref, p1_ref, p2_ref, p3_ref,
                   q0_ref, q1_ref, q2_ref, q3_ref,
                   wh0h, wh0m, bh0, wh1, bh1,
                   wi, bi, wj, hn_ref, xn_ref, t1_ref, t2_ref):
    h = h_ref[...]
    magg = (p0_ref[...] + p1_ref[...]) + (p2_ref[...] + p3_ref[...])
    xn_ref[...] = x_ref[...] + ((q0_ref[...] + q1_ref[...])
                                + (q2_ref[...] + q3_ref[...]))
    u = _silu(_dot(h, wh0h[...]) + _dot(magg, wh0m[...]) + bh0[...])
    hn = _dot(u, wh1[...]) + bh1[...]
    hn_ref[...] = hn
    t1_ref[...] = _dot(hn, wi[...]) + bi[...]
    t2_ref[...] = _dot(hn, wj[...])


def _node_last_body(h_ref, p0_ref, p1_ref, p2_ref, p3_ref,
                    wh0h, wh0m, bh0, wh1, bh1, hn_ref):
    h = h_ref[...]
    magg = (p0_ref[...] + p1_ref[...]) + (p2_ref[...] + p3_ref[...])
    u = _silu(_dot(h, wh0h[...]) + _dot(magg, wh0m[...]) + bh0[...])
    hn_ref[...] = _dot(u, wh1[...]) + bh1[...]


def _readout_body(h_ref, b_ref, w0, b0, w1, b1, wp0, bp0, wp1, bp1,
                  sums_ref, cnts_ref, out_ref):
    i = pl.program_id(0)

    @pl.when(i == 0)
    def _():
        sums_ref[...] = jnp.zeros_like(sums_ref)
        cnts_ref[...] = jnp.zeros_like(cnts_ref)
        out_ref[...] = jnp.zeros_like(out_ref)

    t = _silu(_dot(h_ref[...], w0[...]) + b0[...])
    t = _dot(t, w1[...]) + b1[...]
    og = (b_ref[...] == lax.broadcasted_iota(jnp.int32, (BN, NG), 1)
          ).astype(jnp.float32)
    cdims = (((0,), (0,)), ((), ()))
    sums_ref[...] += lax.dot_general(og, t, cdims,
                                     preferred_element_type=jnp.float32,
                                     precision=lax.Precision.HIGHEST)
    cnts_ref[...] += lax.dot_general(og, jnp.ones((BN, HID), jnp.float32),
                                     cdims, preferred_element_type=jnp.float32,
                                     precision=lax.Precision.HIGHEST)

    @pl.when(i == pl.num_programs(0) - 1)
    def _():
        hg = sums_ref[...] / jnp.maximum(cnts_ref[...], 1.0)
        z = _silu(_dot(hg, wp0[...]) + bp0[...])
        out_ref[...] = (jnp.sum(z * wp1[...], axis=1, keepdims=True)
                        + bp1[...])


def _node_init_call(na, wemb, bemb, wi, bi, wj):
    grid = (N // BN,)
    return pl.pallas_call(
        _node_init_body,
        grid=grid,
        in_specs=[_rows((BN, HID)),
                  _full((HID, HID)), _full((1, HID)),
                  _full((HID, HID)), _full((1, HID)), _full((HID, HID))],
        out_specs=[_rows((BN, HID)), _rows((BN, HID)), _rows((BN, HID))],
        out_shape=[jax.ShapeDtypeStruct((N, HID), jnp.float32),
                   jax.ShapeDtypeStruct((N, HID), jnp.float32),
                   jax.ShapeDtypeStruct((N, HID), jnp.float32)],
    )(na, wemb, bemb, wi, bi, wj)


def _edge_call(g, d, ea, w, be):
    ne = g.shape[0]
    grid = (ne // be,)
    return pl.pallas_call(
        _edge_body,
        grid=grid,
        in_specs=[_rows((be, HID)), _rows((be, XW)), _rows((be, EDIM)),
                  _full((EDIM, HID)), _full((1, HID)),
                  _full((HID, HID)), _full((1, HID)),
                  _full((1, HID)), _full((1, 1)),
                  _full((HID, HID)), _full((1, HID)),
                  _full((1, HID)), _full((1, 1))],
        out_specs=[_rows((be, HID)), _rows((be, XW))],
        out_shape=[jax.ShapeDtypeStruct((ne, HID), jnp.float32),
                   jax.ShapeDtypeStruct((ne, XW), jnp.float32)],
    )(g, d, ea, w["we"], w["wr"], w["we1"], w["be1"], w["winf"],
      w["binf"], w["wx0"], w["bx0"], w["wx1"], w["bx1"])


def _node_mid_call(h, x16, ps, qs, w, wi, bi, wj):
    grid = (N // BN,)
    return pl.pallas_call(
        _node_mid_body,
        grid=grid,
        in_specs=[_rows((BN, HID)), _rows((BN, XW))]
                 + [_rows((BN, HID))] * 4 + [_rows((BN, XW))] * 4
                 + [_full((HID, HID)), _full((HID, HID)), _full((1, HID)),
                    _full((HID, HID)), _full((1, HID)),
                    _full((HID, HID)), _full((1, HID)), _full((HID, HID))],
        out_specs=[_rows((BN, HID)), _rows((BN, XW)),
                   _rows((BN, HID)), _rows((BN, HID))],
        out_shape=[jax.ShapeDtypeStruct((N, HID), jnp.float32),
                   jax.ShapeDtypeStruct((N, XW), jnp.float32),
                   jax.ShapeDtypeStruct((N, HID), jnp.float32),
                   jax.ShapeDtypeStruct((N, HID), jnp.float32)],
    )(h, x16, *ps, *qs, w["wh0h"], w["wh0m"], w["bh0"], w["wh1"],
      w["bh1"], wi, bi, wj)


def _node_last_call(h, ps, w):
    grid = (N // BN,)
    return pl.pallas_call(
        _node_last_body,
        grid=grid,
        in_specs=[_rows((BN, HID))] + [_rows((BN, HID))] * 4
                 + [_full((HID, HID)), _full((HID, HID)), _full((1, HID)),
                    _full((HID, HID)), _full((1, HID))],
        out_specs=[_rows((BN, HID))],
        out_shape=[jax.ShapeDtypeStruct((N, HID), jnp.float32)],
    )(h, *ps, w["wh0h"], w["wh0m"], w["bh0"], w["wh1"], w["bh1"])[0]


def _readout_call(h, bids, w):
    grid = (N // BN,)
    return pl.pallas_call(
        _readout_body,
        grid=grid,
        in_specs=[_rows((BN, HID)), _rows((BN, 1)),
                  _full((HID, HID)), _full((1, HID)),
                  _full((HID, HID)), _full((1, HID)),
                  _full((HID, HID)), _full((1, HID)),
                  _full((1, HID)), _full((1, 1))],
        out_specs=[_full((NG, HID)), _full((NG, HID)), _full((NG, 1))],
        out_shape=[jax.ShapeDtypeStruct((NG, HID), jnp.float32),
                   jax.ShapeDtypeStruct((NG, HID), jnp.float32),
                   jax.ShapeDtypeStruct((NG, 1), jnp.float32)],
    )(h, bids, w["w0"], w["b0"], w["w1"], w["b1"],
      w["wp0"], w["bp0"], w["wp1"], w["bp1"])[2]


def _row(v):
    return v.reshape(1, -1)


def kernel(node_attr, pos, edge_attr, params, edge_index, batch):
    src = edge_index[0]
    dst = edge_index[1]
    x16 = jnp.pad(pos, ((0, 0), (0, XW - 3)))
    dst3 = dst.reshape(NTILES, NCH, CHUNK)
    src3 = src.reshape(NTILES, NCH, CHUNK)
    ea4 = edge_attr.reshape(NTILES, NCH, CHUNK, EDIM)
    NA = 62
    halves = []
    for lo, hi, be in ((0, NA, 3968), (NA, NCH, 4032)):
        nch = hi - lo
        halves.append({
            "nch": nch, "be": be,
            "dst3": dst3[:, lo:hi],
            "src3": src3[:, lo:hi],
            "ea": ea4[:, lo:hi].reshape(NTILES * nch * CHUNK, EDIM),
        })
    zeros_m = jnp.zeros((N, HID), jnp.float32)
    zeros_x = jnp.zeros((N, XW), jnp.float32)
    bids = batch.reshape(N, 1)

    layers = params["layers"]

    def e0_split(lp):
        w = lp["e0"]["w"]
        return (w[:HID], _row(lp["e0"]["b"]), w[HID:2 * HID],
                w[2 * HID:2 * HID + 1], w[2 * HID + 1:])

    def edge_w(lp):
        _, _, _, wr, we = e0_split(lp)
        return {
            "we": we, "wr": wr,
            "we1": lp["e1"]["w"], "be1": _row(lp["e1"]["b"]),
            "winf": _row(lp["inf"]["w"][:, 0]),
            "binf": lp["inf"]["b"].reshape(1, 1),
            "wx0": lp["x0"]["w"], "bx0": _row(lp["x0"]["b"]),
            "wx1": _row(lp["x1"]["w"][:, 0]),
            "bx1": lp["x1"]["b"].reshape(1, 1),
        }

    def node_w(lp):
        return {
            "wh0h": lp["h0"]["w"][:HID], "wh0m": lp["h0"]["w"][HID:],
            "bh0": _row(lp["h0"]["b"]),
            "wh1": lp["h1"]["w"], "bh1": _row(lp["h1"]["b"]),
        }

    wi0, bi0, wj0, _, _ = e0_split(layers[0])
    h, t1, t2 = _node_init_call(node_attr, params["embedding"]["w"],
                                _row(params["embedding"]["b"]), wi0, bi0, wj0)

    for l in range(len(layers)):
        lp = layers[l]
        ew = edge_w(lp)
        ps, qs = [], []
        mdx = []
        for hv in halves:
            sc = _sc_kernels(hv["nch"])
            g = sc["gather_f"](t1, t2, hv["dst3"], hv["src3"])
            d = sc["gather_x"](x16, x16, hv["dst3"], hv["src3"])
            mdx.append(_edge_call(g, d, hv["ea"], ew, hv["be"]))
        for hv, (m, dx) in zip(halves, mdx):
            sc = _sc_kernels(hv["nch"])
            p = sc["scatter_m"](m, hv["dst3"], zeros_m).reshape(2, N, HID)
            q = sc["scatter_x"](dx, hv["dst3"], zeros_x).reshape(2, N, XW)
            ps += [p[0], p[1]]
            qs += [q[0], q[1]]
        if l + 1 < len(layers):
            win, bin_, wjn, _, _ = e0_split(layers[l + 1])
            h, x16, t1, t2 = _node_mid_call(h, x16, ps, qs,
                                            node_w(lp), win, bin_, wjn)
        else:
            h = _node_last_call(h, ps, node_w(lp))

    rw = {
        "w0": params["lin0"]["w"], "b0": _row(params["lin0"]["b"]),
        "w1": params["lin1"]["w"], "b1": _row(params["lin1"]["b"]),
        "wp0": params["pred0"]["w"], "bp0": _row(params["pred0"]["b"]),
        "wp1": _row(params["pred1"]["w"][:, 0]),
        "bp1": params["pred1"]["b"].reshape(1, 1),
    }
    out = _readout_call(h, bids, rw)
    return out.reshape(-1)
